# baseline (speedup 1.0000x reference)
import functools

import jax
import jax.numpy as jnp
from jax import lax
from jax.experimental import pallas as pl
from jax.experimental.pallas import tpu as pltpu
from jax.experimental.pallas import tpu_sc as plsc

_MODS = ('LM', 'VIS', 'CLIP', 'MAE')
_DIMS = {'LM': 4096, 'VIS': 768, 'CLIP': 512, 'MAE': 1024}
_H = 512
_CD = 256
_NC = 8192
_B = 4096
_RB = 512
_CB = 512
_EPS = 1e-5


def _tree_rowsum(x):
    n = x.shape[1]
    acc = x[:, 0:8]
    for v in range(8, n, 8):
        acc = acc + x[:, v:v + 8]
    for sh in (4, 2, 1):
        acc = acc + jnp.roll(acc, -sh, axis=1)
    return acc[:, 0:1]


def _comp_rowsum(x):
    n = x.shape[1]
    s = x
    e = jnp.zeros_like(x)
    while n > 1:
        h = n // 2
        a, b = s[:, :h], s[:, h:n]
        t = a + b
        bp = t - a
        err = (a - (t - bp)) + (b - bp)
        e = e[:, :h] + e[:, h:n] + err
        s = t
        n = h
    return s[:, :1] + e[:, :1]


def _dot256(a, b):
    acc = lax.dot_general(a[:, 0:256], b[:, 0:256], (((1,), (1,)), ((), ())),
                          preferred_element_type=jnp.float32)
    for k in range(256, a.shape[1], 256):
        acc = acc + lax.dot_general(a[:, k:k + 256], b[:, k:k + 256],
                                    (((1,), (1,)), ((), ())),
                                    preferred_element_type=jnp.float32)
    return acc


def _ln(h, g, b):
    n = h.shape[1]
    mu = _tree_rowsum(h) / n
    t = h - mu
    var = _tree_rowsum(t * t) / n
    return t / jnp.sqrt(var + _EPS) * g + b


def _mm_body(x_ref, w_ref, b_ref, o_ref):
    o_ref[...] = _dot256(x_ref[...], w_ref[...]) + b_ref[...]


def _mid_body(h1_ref, g1_ref, be1_ref, w2_ref, b2_ref, g2_ref, be2_ref,
              z_ref):
    r1 = jnp.maximum(_ln(h1_ref[...], g1_ref[...], be1_ref[...]), 0.0)
    h2 = _dot256(r1, w2_ref[...]) + b2_ref[...]
    z_ref[...] = _ln(h2, g2_ref[...], be2_ref[...])


@functools.lru_cache(maxsize=None)
def _mm_call(k, n):
    return pl.pallas_call(
        _mm_body,
        grid=(_B // _RB,),
        in_specs=[pl.BlockSpec((_RB, k), lambda i: (i, 0)),
                  pl.BlockSpec((n, k), lambda i: (0, 0)),
                  pl.BlockSpec((1, n), lambda i: (0, 0))],
        out_specs=pl.BlockSpec((_RB, n), lambda i: (i, 0)),
        out_shape=jax.ShapeDtypeStruct((_B, n), jnp.float32),
    )


_mid_call = pl.pallas_call(
    _mid_body,
    grid=(_B // _RB,),
    in_specs=[pl.BlockSpec((_RB, _H), lambda i: (i, 0)),
              pl.BlockSpec((1, _H), lambda i: (0, 0)),
              pl.BlockSpec((1, _H), lambda i: (0, 0)),
              pl.BlockSpec((_CD, _H), lambda i: (0, 0)),
              pl.BlockSpec((1, _CD), lambda i: (0, 0)),
              pl.BlockSpec((1, _CD), lambda i: (0, 0)),
              pl.BlockSpec((1, _CD), lambda i: (0, 0))],
    out_specs=pl.BlockSpec((_RB, _CD), lambda i: (i, 0)),
    out_shape=jax.ShapeDtypeStruct((_B, _CD), jnp.float32),
)


def _dist_body(z_ref, e_ref, idx_ref):
    z = z_ref[...]
    z2 = _tree_rowsum(z * z)

    def step(c, carry):
        best_d, best_i = carry
        eb = e_ref[pl.ds(c * _CB, _CB), :]
        s = lax.dot_general(z, eb, (((1,), (1,)), ((), ())),
                            preferred_element_type=jnp.float32)
        esq = _comp_rowsum(eb * eb).reshape(1, _CB)
        dist = (z2 - 2.0 * s) + esq
        m = jnp.min(dist, axis=1, keepdims=True)
        lane = lax.broadcasted_iota(jnp.int32, dist.shape, 1)
        a = jnp.min(jnp.where(dist == m, lane, 2 ** 30),
                    axis=1)[:, None] + c * _CB
        upd = m < best_d
        return jnp.where(upd, m, best_d), jnp.where(upd, a, best_i)

    init = (jnp.full((_RB, 1), jnp.inf, jnp.float32),
            jnp.zeros((_RB, 1), jnp.int32))
    _, best_i = lax.fori_loop(0, _NC // _CB, step, init)
    idx_ref[...] = best_i


_dist_call = pl.pallas_call(
    _dist_body,
    grid=(_B // _RB,),
    in_specs=[pl.BlockSpec((_RB, _CD), lambda i: (i, 0)),
              pl.BlockSpec((_NC, _CD), lambda i: (0, 0))],
    out_specs=pl.BlockSpec((_RB, 1), lambda i: (i, 0)),
    out_shape=jax.ShapeDtypeStruct((_B, 1), jnp.int32),
)


@functools.lru_cache(maxsize=None)
def _get_gather():
    info = plsc.get_sparse_core_info()
    ncores, nsub = info.num_cores, info.num_subcores
    nw = ncores * nsub
    bpw = _B // nw
    mesh = plsc.VectorSubcoreMesh(core_axis_name="c", subcore_axis_name="s")

    @functools.partial(
        pl.kernel, mesh=mesh,
        out_type=jax.ShapeDtypeStruct((_B, _CD), jnp.float32),
        scratch_types=[
            pltpu.VMEM((bpw,), jnp.int32),
            pltpu.VMEM((bpw, _CD), jnp.float32),
            pltpu.SemaphoreType.DMA,
        ],
    )
    def gather(table_hbm, idx_hbm, out_hbm, idx_v, rows_v, sem):
        wid = lax.axis_index("s") * ncores + lax.axis_index("c")
        base = wid * bpw
        pltpu.sync_copy(idx_hbm.at[pl.ds(base, bpw)], idx_v)
        pltpu.async_copy(table_hbm.at[idx_v], rows_v, sem).wait()
        pltpu.sync_copy(rows_v, out_hbm.at[pl.ds(base, bpw)])

    return gather


def _recon_body(z_ref, q_ref, wd_ref, bd_ref, qst_ref, rec_ref, com_ref):
    i = pl.program_id(0)
    z = z_ref[...]
    q = q_ref[...]
    qst = z + (q - z)
    qst_ref[...] = qst
    rec_ref[...] = _dot256(qst, wd_ref[...]) + bd_ref[...]

    @pl.when(i == 0)
    def _():
        com_ref[...] = jnp.zeros((1, 1), jnp.float32)

    com_ref[...] += jnp.sum((z - q) ** 2, keepdims=True)

    @pl.when(i == pl.num_programs(0) - 1)
    def _():
        com_ref[...] = com_ref[...] / (_B * _CD)


@functools.lru_cache(maxsize=None)
def _recon_call(d):
    return pl.pallas_call(
        _recon_body,
        grid=(_B // _RB,),
        in_specs=[pl.BlockSpec((_RB, _CD), lambda i: (i, 0)),
                  pl.BlockSpec((_RB, _CD), lambda i: (i, 0)),
                  pl.BlockSpec((d, _CD), lambda i: (0, 0)),
                  pl.BlockSpec((1, d), lambda i: (0, 0))],
        out_specs=[pl.BlockSpec((_RB, _CD), lambda i: (i, 0)),
                   pl.BlockSpec((_RB, d), lambda i: (i, 0)),
                   pl.BlockSpec((1, 1), lambda i: (0, 0))],
        out_shape=[jax.ShapeDtypeStruct((_B, _CD), jnp.float32),
                   jax.ShapeDtypeStruct((_B, d), jnp.float32),
                   jax.ShapeDtypeStruct((1, 1), jnp.float32)],
    )


def kernel(LM, VIS, CLIP, MAE, params):
    xs = {'LM': LM, 'VIS': VIS, 'CLIP': CLIP, 'MAE': MAE}
    E = params['codebook']
    gather = _get_gather()
    out = {}
    for m in _MODS:
        p = params[m]
        d = _DIMS[m]
        h1 = _mm_call(d, _H)(xs[m], p['W1'], p['b1'].reshape(1, _H))
        z = _mid_call(h1, p['g1'].reshape(1, _H), p['be1'].reshape(1, _H),
                      p['W2'], p['b2'].reshape(1, _CD),
                      p['g2'].reshape(1, _CD), p['be2'].reshape(1, _CD))
        idx = _dist_call(z, E).reshape(_B)
        q = gather(E, idx)
        qst, rec, com = _recon_call(d)(z, q, p['Wd'], p['bd'].reshape(1, d))
        out[f'{m}_z'] = z
        out[f'{m}_q'] = qst
        out[f'{m}_idx'] = idx
        out[f'{m}_commit'] = com.reshape(())
        out[f'{m}_recon'] = rec
    return out


# speedup vs baseline: 1.7616x; 1.7616x over previous
import functools

import jax
import jax.numpy as jnp
from jax import lax
from jax.experimental import pallas as pl
from jax.experimental.pallas import tpu as pltpu
from jax.experimental.pallas import tpu_sc as plsc

_MODS = ('LM', 'VIS', 'CLIP', 'MAE')
_DIMS = {'LM': 4096, 'VIS': 768, 'CLIP': 512, 'MAE': 1024}
_H = 512
_CD = 256
_NC = 8192
_B = 4096
_RB = 512
_CB = 512
_EPS = 1e-5


def _tree_rowsum(x):
    n = x.shape[1]
    acc = x[:, 0:8]
    for v in range(8, n, 8):
        acc = acc + x[:, v:v + 8]
    for sh in (4, 2, 1):
        acc = acc + jnp.roll(acc, -sh, axis=1)
    return acc[:, 0:1]


def _comp_rowsum(x):
    n = x.shape[1]
    s = x
    e = jnp.zeros_like(x)
    while n > 1:
        h = n // 2
        a, b = s[:, :h], s[:, h:n]
        t = a + b
        bp = t - a
        err = (a - (t - bp)) + (b - bp)
        e = e[:, :h] + e[:, h:n] + err
        s = t
        n = h
    return s[:, :1] + e[:, :1]


def _dot256(a, b):
    acc = lax.dot_general(a[:, 0:256], b[:, 0:256], (((1,), (1,)), ((), ())),
                          preferred_element_type=jnp.float32)
    for k in range(256, a.shape[1], 256):
        acc = acc + lax.dot_general(a[:, k:k + 256], b[:, k:k + 256],
                                    (((1,), (1,)), ((), ())),
                                    preferred_element_type=jnp.float32)
    return acc


def _ln(h, g, b):
    n = h.shape[1]
    mu = _tree_rowsum(h) / n
    t = h - mu
    var = _tree_rowsum(t * t) / n
    return t / jnp.sqrt(var + _EPS) * g + b


def _mm_body(x_ref, w_ref, b_ref, o_ref):
    o_ref[...] = _dot256(x_ref[...], w_ref[...]) + b_ref[...]


def _mid_body(h1_ref, g1_ref, be1_ref, w2_ref, b2_ref, g2_ref, be2_ref,
              z_ref):
    r1 = jnp.maximum(_ln(h1_ref[...], g1_ref[...], be1_ref[...]), 0.0)
    h2 = _dot256(r1, w2_ref[...]) + b2_ref[...]
    z_ref[...] = _ln(h2, g2_ref[...], be2_ref[...])


@functools.lru_cache(maxsize=None)
def _mm_call(k, n):
    return pl.pallas_call(
        _mm_body,
        grid=(_B // _RB,),
        in_specs=[pl.BlockSpec((_RB, k), lambda i: (i, 0)),
                  pl.BlockSpec((n, k), lambda i: (0, 0)),
                  pl.BlockSpec((1, n), lambda i: (0, 0))],
        out_specs=pl.BlockSpec((_RB, n), lambda i: (i, 0)),
        out_shape=jax.ShapeDtypeStruct((_B, n), jnp.float32),
    )


_mid_call = pl.pallas_call(
    _mid_body,
    grid=(_B // _RB,),
    in_specs=[pl.BlockSpec((_RB, _H), lambda i: (i, 0)),
              pl.BlockSpec((1, _H), lambda i: (0, 0)),
              pl.BlockSpec((1, _H), lambda i: (0, 0)),
              pl.BlockSpec((_CD, _H), lambda i: (0, 0)),
              pl.BlockSpec((1, _CD), lambda i: (0, 0)),
              pl.BlockSpec((1, _CD), lambda i: (0, 0)),
              pl.BlockSpec((1, _CD), lambda i: (0, 0))],
    out_specs=pl.BlockSpec((_RB, _CD), lambda i: (i, 0)),
    out_shape=jax.ShapeDtypeStruct((_B, _CD), jnp.float32),
)


def _esq_body(e_ref, o_ref):
    eb = e_ref[...]
    o_ref[...] = _comp_rowsum(eb * eb)


_esq_call = pl.pallas_call(
    _esq_body,
    grid=(_NC // _RB,),
    in_specs=[pl.BlockSpec((_RB, _CD), lambda i: (i, 0))],
    out_specs=pl.BlockSpec((_RB, 1), lambda i: (i, 0)),
    out_shape=jax.ShapeDtypeStruct((_NC, 1), jnp.float32),
)


def _dist_body(z_ref, e_ref, esq_ref, idx_ref):
    z = z_ref[...]
    z2 = _tree_rowsum(z * z)

    def step(c, carry):
        best_d, best_i = carry
        eb = e_ref[pl.ds(c * _CB, _CB), :]
        s = lax.dot_general(z, eb, (((1,), (1,)), ((), ())),
                            preferred_element_type=jnp.float32)
        esq = esq_ref[pl.ds(c * _CB, _CB), :].reshape(1, _CB)
        dist = (z2 - 2.0 * s) + esq
        m = jnp.min(dist, axis=1, keepdims=True)
        lane = lax.broadcasted_iota(jnp.int32, dist.shape, 1)
        a = jnp.min(jnp.where(dist == m, lane, 2 ** 30),
                    axis=1)[:, None] + c * _CB
        upd = m < best_d
        return jnp.where(upd, m, best_d), jnp.where(upd, a, best_i)

    init = (jnp.full((_RB, 1), jnp.inf, jnp.float32),
            jnp.zeros((_RB, 1), jnp.int32))
    _, best_i = lax.fori_loop(0, _NC // _CB, step, init)
    idx_ref[...] = best_i


_dist_call = pl.pallas_call(
    _dist_body,
    grid=(_B // _RB,),
    in_specs=[pl.BlockSpec((_RB, _CD), lambda i: (i, 0)),
              pl.BlockSpec((_NC, _CD), lambda i: (0, 0)),
              pl.BlockSpec((_NC, 1), lambda i: (0, 0))],
    out_specs=pl.BlockSpec((_RB, 1), lambda i: (i, 0)),
    out_shape=jax.ShapeDtypeStruct((_B, 1), jnp.int32),
)


@functools.lru_cache(maxsize=None)
def _get_gather():
    info = plsc.get_sparse_core_info()
    ncores, nsub = info.num_cores, info.num_subcores
    nw = ncores * nsub
    bpw = _B // nw
    mesh = plsc.VectorSubcoreMesh(core_axis_name="c", subcore_axis_name="s")

    @functools.partial(
        pl.kernel, mesh=mesh,
        out_type=jax.ShapeDtypeStruct((_B, _CD), jnp.float32),
        scratch_types=[
            pltpu.VMEM((bpw,), jnp.int32),
            pltpu.VMEM((bpw, _CD), jnp.float32),
            pltpu.SemaphoreType.DMA,
        ],
    )
    def gather(table_hbm, idx_hbm, out_hbm, idx_v, rows_v, sem):
        wid = lax.axis_index("s") * ncores + lax.axis_index("c")
        base = wid * bpw
        pltpu.sync_copy(idx_hbm.at[pl.ds(base, bpw)], idx_v)
        pltpu.async_copy(table_hbm.at[idx_v], rows_v, sem).wait()
        pltpu.sync_copy(rows_v, out_hbm.at[pl.ds(base, bpw)])

    return gather


def _recon_body(z_ref, q_ref, wd_ref, bd_ref, qst_ref, rec_ref, com_ref):
    i = pl.program_id(0)
    z = z_ref[...]
    q = q_ref[...]
    qst = z + (q - z)
    qst_ref[...] = qst
    rec_ref[...] = _dot256(qst, wd_ref[...]) + bd_ref[...]

    @pl.when(i == 0)
    def _():
        com_ref[...] = jnp.zeros((1, 1), jnp.float32)

    com_ref[...] += jnp.sum((z - q) ** 2, keepdims=True)

    @pl.when(i == pl.num_programs(0) - 1)
    def _():
        com_ref[...] = com_ref[...] / (_B * _CD)


@functools.lru_cache(maxsize=None)
def _recon_call(d):
    return pl.pallas_call(
        _recon_body,
        grid=(_B // _RB,),
        in_specs=[pl.BlockSpec((_RB, _CD), lambda i: (i, 0)),
                  pl.BlockSpec((_RB, _CD), lambda i: (i, 0)),
                  pl.BlockSpec((d, _CD), lambda i: (0, 0)),
                  pl.BlockSpec((1, d), lambda i: (0, 0))],
        out_specs=[pl.BlockSpec((_RB, _CD), lambda i: (i, 0)),
                   pl.BlockSpec((_RB, d), lambda i: (i, 0)),
                   pl.BlockSpec((1, 1), lambda i: (0, 0))],
        out_shape=[jax.ShapeDtypeStruct((_B, _CD), jnp.float32),
                   jax.ShapeDtypeStruct((_B, d), jnp.float32),
                   jax.ShapeDtypeStruct((1, 1), jnp.float32)],
    )


def kernel(LM, VIS, CLIP, MAE, params):
    xs = {'LM': LM, 'VIS': VIS, 'CLIP': CLIP, 'MAE': MAE}
    E = params['codebook']
    gather = _get_gather()
    esq = _esq_call(E)
    out = {}
    for m in _MODS:
        p = params[m]
        d = _DIMS[m]
        h1 = _mm_call(d, _H)(xs[m], p['W1'], p['b1'].reshape(1, _H))
        z = _mid_call(h1, p['g1'].reshape(1, _H), p['be1'].reshape(1, _H),
                      p['W2'], p['b2'].reshape(1, _CD),
                      p['g2'].reshape(1, _CD), p['be2'].reshape(1, _CD))
        idx = _dist_call(z, E, esq).reshape(_B)
        q = gather(E, idx)
        qst, rec, com = _recon_call(d)(z, q, p['Wd'], p['bd'].reshape(1, d))
        out[f'{m}_z'] = z
        out[f'{m}_q'] = qst
        out[f'{m}_idx'] = idx
        out[f'{m}_commit'] = com.reshape(())
        out[f'{m}_recon'] = rec
    return out


# speedup vs baseline: 1.9022x; 1.0798x over previous
import functools

import jax
import jax.numpy as jnp
from jax import lax
from jax.experimental import pallas as pl
from jax.experimental.pallas import tpu as pltpu
from jax.experimental.pallas import tpu_sc as plsc

_MODS = ('LM', 'VIS', 'CLIP', 'MAE')
_DIMS = {'LM': 4096, 'VIS': 768, 'CLIP': 512, 'MAE': 1024}
_H = 512
_CD = 256
_NC = 8192
_B = 4096
_RB = 512
_CB = 512
_EPS = 1e-5


def _tree_rowsum(x):
    n = x.shape[1]
    acc = x[:, 0:8]
    for v in range(8, n, 8):
        acc = acc + x[:, v:v + 8]
    for sh in (4, 2, 1):
        acc = acc + jnp.roll(acc, -sh, axis=1)
    return acc[:, 0:1]


def _comp_rowsum(x):
    n = x.shape[1]
    s = x
    e = jnp.zeros_like(x)
    while n > 1:
        h = n // 2
        a, b = s[:, :h], s[:, h:n]
        t = a + b
        bp = t - a
        err = (a - (t - bp)) + (b - bp)
        e = e[:, :h] + e[:, h:n] + err
        s = t
        n = h
    return s[:, :1] + e[:, :1]


def _dot256(a, b):
    acc = lax.dot_general(a[:, 0:256], b[:, 0:256], (((1,), (1,)), ((), ())),
                          preferred_element_type=jnp.float32)
    for k in range(256, a.shape[1], 256):
        acc = acc + lax.dot_general(a[:, k:k + 256], b[:, k:k + 256],
                                    (((1,), (1,)), ((), ())),
                                    preferred_element_type=jnp.float32)
    return acc


def _ln(h, g, b):
    n = h.shape[1]
    mu = _tree_rowsum(h) / n
    t = h - mu
    var = _tree_rowsum(t * t) / n
    return t / jnp.sqrt(var + _EPS) * g + b


def _mm_body(x_ref, w_ref, b_ref, o_ref):
    o_ref[...] = _dot256(x_ref[...], w_ref[...]) + b_ref[...]


@functools.lru_cache(maxsize=None)
def _mm_call(k, n):
    return pl.pallas_call(
        _mm_body,
        grid=(_B // _RB,),
        in_specs=[pl.BlockSpec((_RB, k), lambda i: (i, 0)),
                  pl.BlockSpec((n, k), lambda i: (0, 0)),
                  pl.BlockSpec((1, n), lambda i: (0, 0))],
        out_specs=pl.BlockSpec((_RB, n), lambda i: (i, 0)),
        out_shape=jax.ShapeDtypeStruct((_B, n), jnp.float32),
    )


def _mid4_body(h1_ref, g1_ref, be1_ref, w2_ref, b2_ref, g2_ref, be2_ref,
               z_ref):
    r1 = jnp.maximum(_ln(h1_ref[...], g1_ref[0], be1_ref[0]), 0.0)
    h2 = _dot256(r1, w2_ref[0]) + b2_ref[0]
    z_ref[...] = _ln(h2, g2_ref[0], be2_ref[0])


_NB = _B // _RB

_mid4_call = pl.pallas_call(
    _mid4_body,
    grid=(4 * _NB,),
    in_specs=[pl.BlockSpec((_RB, _H), lambda i: (i, 0)),
              pl.BlockSpec((1, 1, _H), lambda i: (i // _NB, 0, 0)),
              pl.BlockSpec((1, 1, _H), lambda i: (i // _NB, 0, 0)),
              pl.BlockSpec((1, _CD, _H), lambda i: (i // _NB, 0, 0)),
              pl.BlockSpec((1, 1, _CD), lambda i: (i // _NB, 0, 0)),
              pl.BlockSpec((1, 1, _CD), lambda i: (i // _NB, 0, 0)),
              pl.BlockSpec((1, 1, _CD), lambda i: (i // _NB, 0, 0))],
    out_specs=pl.BlockSpec((_RB, _CD), lambda i: (i, 0)),
    out_shape=jax.ShapeDtypeStruct((4 * _B, _CD), jnp.float32),
)


def _esq_body(e_ref, o_ref):
    eb = e_ref[...]
    o_ref[...] = _comp_rowsum(eb * eb)


_esq_call = pl.pallas_call(
    _esq_body,
    grid=(_NC // _RB,),
    in_specs=[pl.BlockSpec((_RB, _CD), lambda i: (i, 0))],
    out_specs=pl.BlockSpec((_RB, 1), lambda i: (i, 0)),
    out_shape=jax.ShapeDtypeStruct((_NC, 1), jnp.float32),
)


def _dist_body(z_ref, e_ref, esq_ref, idx_ref):
    z = z_ref[...]
    z2 = _tree_rowsum(z * z)

    def step(c, carry):
        best_d, best_i = carry
        eb = e_ref[pl.ds(c * _CB, _CB), :]
        s = lax.dot_general(z, eb, (((1,), (1,)), ((), ())),
                            preferred_element_type=jnp.float32)
        esq = esq_ref[pl.ds(c * _CB, _CB), :].reshape(1, _CB)
        dist = (z2 - 2.0 * s) + esq
        m = jnp.min(dist, axis=1, keepdims=True)
        lane = lax.broadcasted_iota(jnp.int32, dist.shape, 1)
        a = jnp.min(jnp.where(dist == m, lane, 2 ** 30),
                    axis=1)[:, None] + c * _CB
        upd = m < best_d
        return jnp.where(upd, m, best_d), jnp.where(upd, a, best_i)

    init = (jnp.full((_RB, 1), jnp.inf, jnp.float32),
            jnp.zeros((_RB, 1), jnp.int32))
    _, best_i = lax.fori_loop(0, _NC // _CB, step, init)
    idx_ref[...] = best_i


_dist_call = pl.pallas_call(
    _dist_body,
    grid=(4 * _B // _RB,),
    in_specs=[pl.BlockSpec((_RB, _CD), lambda i: (i, 0)),
              pl.BlockSpec((_NC, _CD), lambda i: (0, 0)),
              pl.BlockSpec((_NC, 1), lambda i: (0, 0))],
    out_specs=pl.BlockSpec((_RB, 1), lambda i: (i, 0)),
    out_shape=jax.ShapeDtypeStruct((4 * _B, 1), jnp.int32),
)


@functools.lru_cache(maxsize=None)
def _get_gather():
    info = plsc.get_sparse_core_info()
    ncores, nsub = info.num_cores, info.num_subcores
    nw = ncores * nsub
    bpw = (4 * _B) // nw
    nchunk = 4
    cs = bpw // nchunk
    mesh = plsc.VectorSubcoreMesh(core_axis_name="c", subcore_axis_name="s")

    @functools.partial(
        pl.kernel, mesh=mesh,
        out_type=jax.ShapeDtypeStruct((4 * _B, _CD), jnp.float32),
        scratch_types=[
            pltpu.VMEM((cs,), jnp.int32),
            pltpu.VMEM((cs, _CD), jnp.float32),
            pltpu.SemaphoreType.DMA,
        ],
    )
    def gather(table_hbm, idx_hbm, out_hbm, idx_v, rows_v, sem):
        wid = lax.axis_index("s") * ncores + lax.axis_index("c")
        base = wid * bpw
        for t in range(nchunk):
            off = base + t * cs
            pltpu.sync_copy(idx_hbm.at[pl.ds(off, cs)], idx_v)
            pltpu.async_copy(table_hbm.at[idx_v], rows_v, sem).wait()
            pltpu.sync_copy(rows_v, out_hbm.at[pl.ds(off, cs)])

    return gather


def _recon_body(z_ref, q_ref, wd_ref, bd_ref, qst_ref, rec_ref, com_ref):
    i = pl.program_id(0)
    z = z_ref[...]
    q = q_ref[...]
    qst = z + (q - z)
    qst_ref[...] = qst
    rec_ref[...] = _dot256(qst, wd_ref[...]) + bd_ref[...]

    @pl.when(i == 0)
    def _():
        com_ref[...] = jnp.zeros((1, 1), jnp.float32)

    com_ref[...] += jnp.sum((z - q) ** 2, keepdims=True)

    @pl.when(i == pl.num_programs(0) - 1)
    def _():
        com_ref[...] = com_ref[...] / (_B * _CD)


@functools.lru_cache(maxsize=None)
def _recon_call(d):
    return pl.pallas_call(
        _recon_body,
        grid=(_B // _RB,),
        in_specs=[pl.BlockSpec((_RB, _CD), lambda i: (i, 0)),
                  pl.BlockSpec((_RB, _CD), lambda i: (i, 0)),
                  pl.BlockSpec((d, _CD), lambda i: (0, 0)),
                  pl.BlockSpec((1, d), lambda i: (0, 0))],
        out_specs=[pl.BlockSpec((_RB, _CD), lambda i: (i, 0)),
                   pl.BlockSpec((_RB, d), lambda i: (i, 0)),
                   pl.BlockSpec((1, 1), lambda i: (0, 0))],
        out_shape=[jax.ShapeDtypeStruct((_B, _CD), jnp.float32),
                   jax.ShapeDtypeStruct((_B, d), jnp.float32),
                   jax.ShapeDtypeStruct((1, 1), jnp.float32)],
    )


def kernel(LM, VIS, CLIP, MAE, params):
    xs = {'LM': LM, 'VIS': VIS, 'CLIP': CLIP, 'MAE': MAE}
    E = params['codebook']
    gather = _get_gather()
    esq = _esq_call(E)

    h1s = [_mm_call(_DIMS[m], _H)(xs[m], params[m]['W1'],
                                  params[m]['b1'].reshape(1, _H))
           for m in _MODS]
    H1 = jnp.concatenate(h1s, axis=0)
    st = lambda name, n: jnp.stack(
        [params[m][name].reshape(1, n) for m in _MODS])
    Z = _mid4_call(H1, st('g1', _H), st('be1', _H),
                   jnp.stack([params[m]['W2'] for m in _MODS]),
                   st('b2', _CD), st('g2', _CD), st('be2', _CD))
    idx_all = _dist_call(Z, E, esq).reshape(4 * _B)
    Q = gather(E, idx_all)

    out = {}
    for i, m in enumerate(_MODS):
        p = params[m]
        d = _DIMS[m]
        z = Z[i * _B:(i + 1) * _B]
        q = Q[i * _B:(i + 1) * _B]
        qst, rec, com = _recon_call(d)(z, q, p['Wd'], p['bd'].reshape(1, d))
        out[f'{m}_z'] = z
        out[f'{m}_q'] = qst
        out[f'{m}_idx'] = idx_all[i * _B:(i + 1) * _B]
        out[f'{m}_commit'] = com.reshape(())
        out[f'{m}_recon'] = rec
    return out


# speedup vs baseline: 1.9270x; 1.0130x over previous
import functools

import jax
import jax.numpy as jnp
from jax import lax
from jax.experimental import pallas as pl
from jax.experimental.pallas import tpu as pltpu
from jax.experimental.pallas import tpu_sc as plsc

_MODS = ('LM', 'VIS', 'CLIP', 'MAE')
_DIMS = {'LM': 4096, 'VIS': 768, 'CLIP': 512, 'MAE': 1024}
_H = 512
_CD = 256
_NC = 8192
_B = 4096
_RB = 512
_CB = 1024
_EPS = 1e-5


def _tree_rowsum(x):
    n = x.shape[1]
    acc = x[:, 0:8]
    for v in range(8, n, 8):
        acc = acc + x[:, v:v + 8]
    for sh in (4, 2, 1):
        acc = acc + jnp.roll(acc, -sh, axis=1)
    return acc[:, 0:1]


def _comp_rowsum(x):
    n = x.shape[1]
    s = x
    e = jnp.zeros_like(x)
    while n > 1:
        h = n // 2
        a, b = s[:, :h], s[:, h:n]
        t = a + b
        bp = t - a
        err = (a - (t - bp)) + (b - bp)
        e = e[:, :h] + e[:, h:n] + err
        s = t
        n = h
    return s[:, :1] + e[:, :1]


def _dot256(a, b):
    acc = lax.dot_general(a[:, 0:256], b[:, 0:256], (((1,), (1,)), ((), ())),
                          preferred_element_type=jnp.float32)
    for k in range(256, a.shape[1], 256):
        acc = acc + lax.dot_general(a[:, k:k + 256], b[:, k:k + 256],
                                    (((1,), (1,)), ((), ())),
                                    preferred_element_type=jnp.float32)
    return acc


def _ln(h, g, b):
    n = h.shape[1]
    mu = _tree_rowsum(h) / n
    t = h - mu
    var = _tree_rowsum(t * t) / n
    return t / jnp.sqrt(var + _EPS) * g + b


def _mm_body(x_ref, w_ref, b_ref, o_ref):
    o_ref[...] = _dot256(x_ref[...], w_ref[...]) + b_ref[...]


@functools.lru_cache(maxsize=None)
def _mm_call(k, n):
    return pl.pallas_call(
        _mm_body,
        grid=(_B // _RB,),
        in_specs=[pl.BlockSpec((_RB, k), lambda i: (i, 0)),
                  pl.BlockSpec((n, k), lambda i: (0, 0)),
                  pl.BlockSpec((1, n), lambda i: (0, 0))],
        out_specs=pl.BlockSpec((_RB, n), lambda i: (i, 0)),
        out_shape=jax.ShapeDtypeStruct((_B, n), jnp.float32),
    )


_NB = _B // _RB


def _mid4_body(h1_ref, g1_ref, be1_ref, w2_ref, b2_ref, g2_ref, be2_ref,
               e_ref, esq_ref, z_ref, idx_ref):
    r1 = jnp.maximum(_ln(h1_ref[...], g1_ref[0], be1_ref[0]), 0.0)
    h2 = _dot256(r1, w2_ref[0]) + b2_ref[0]
    z = _ln(h2, g2_ref[0], be2_ref[0])
    z_ref[...] = z
    z2 = _tree_rowsum(z * z)

    def step(c, carry):
        best_d, best_i = carry
        eb = e_ref[pl.ds(c * _CB, _CB), :]
        s = lax.dot_general(z, eb, (((1,), (1,)), ((), ())),
                            preferred_element_type=jnp.float32)
        esq = esq_ref[pl.ds(c * _CB, _CB), :].reshape(1, _CB)
        dist = (z2 - 2.0 * s) + esq
        m = jnp.min(dist, axis=1, keepdims=True)
        lane = lax.broadcasted_iota(jnp.int32, dist.shape, 1)
        a = jnp.min(jnp.where(dist == m, lane, 2 ** 30),
                    axis=1)[:, None] + c * _CB
        upd = m < best_d
        return jnp.where(upd, m, best_d), jnp.where(upd, a, best_i)

    init = (jnp.full((_RB, 1), jnp.inf, jnp.float32),
            jnp.zeros((_RB, 1), jnp.int32))
    _, best_i = lax.fori_loop(0, _NC // _CB, step, init)
    idx_ref[...] = best_i


_mid4_call = pl.pallas_call(
    _mid4_body,
    grid=(4 * _NB,),
    in_specs=[pl.BlockSpec((_RB, _H), lambda i: (i, 0)),
              pl.BlockSpec((1, 1, _H), lambda i: (i // _NB, 0, 0)),
              pl.BlockSpec((1, 1, _H), lambda i: (i // _NB, 0, 0)),
              pl.BlockSpec((1, _CD, _H), lambda i: (i // _NB, 0, 0)),
              pl.BlockSpec((1, 1, _CD), lambda i: (i // _NB, 0, 0)),
              pl.BlockSpec((1, 1, _CD), lambda i: (i // _NB, 0, 0)),
              pl.BlockSpec((1, 1, _CD), lambda i: (i // _NB, 0, 0)),
              pl.BlockSpec((_NC, _CD), lambda i: (0, 0)),
              pl.BlockSpec((_NC, 1), lambda i: (0, 0))],
    out_specs=[pl.BlockSpec((_RB, _CD), lambda i: (i, 0)),
               pl.BlockSpec((_RB, 1), lambda i: (i, 0))],
    out_shape=[jax.ShapeDtypeStruct((4 * _B, _CD), jnp.float32),
               jax.ShapeDtypeStruct((4 * _B, 1), jnp.int32)],
)


def _esq_body(e_ref, o_ref):
    eb = e_ref[...]
    o_ref[...] = _comp_rowsum(eb * eb)


_esq_call = pl.pallas_call(
    _esq_body,
    grid=(_NC // _RB,),
    in_specs=[pl.BlockSpec((_RB, _CD), lambda i: (i, 0))],
    out_specs=pl.BlockSpec((_RB, 1), lambda i: (i, 0)),
    out_shape=jax.ShapeDtypeStruct((_NC, 1), jnp.float32),
)


@functools.lru_cache(maxsize=None)
def _get_gather():
    info = plsc.get_sparse_core_info()
    ncores, nsub = info.num_cores, info.num_subcores
    nw = ncores * nsub
    bpw = (4 * _B) // nw
    nchunk = 4
    cs = bpw // nchunk
    mesh = plsc.VectorSubcoreMesh(core_axis_name="c", subcore_axis_name="s")

    @functools.partial(
        pl.kernel, mesh=mesh,
        out_type=jax.ShapeDtypeStruct((4 * _B, _CD), jnp.float32),
        scratch_types=[
            pltpu.VMEM((cs,), jnp.int32),
            pltpu.VMEM((cs, _CD), jnp.float32),
            pltpu.VMEM((cs,), jnp.int32),
            pltpu.VMEM((cs, _CD), jnp.float32),
            pltpu.SemaphoreType.DMA,
            pltpu.SemaphoreType.DMA,
        ],
    )
    def gather(table_hbm, idx_hbm, out_hbm, idx_v, rows_v, idx_v2, rows_v2,
               sem, sem2):
        wid = lax.axis_index("s") * ncores + lax.axis_index("c")
        base = wid * bpw
        bufs = ((idx_v, rows_v, sem), (idx_v2, rows_v2, sem2))
        handles = {}
        for t in range(min(2, nchunk)):
            iv, rv, sm = bufs[t % 2]
            pltpu.sync_copy(idx_hbm.at[pl.ds(base + t * cs, cs)], iv)
            handles[t] = pltpu.async_copy(table_hbm.at[iv], rv, sm)
        for t in range(nchunk):
            iv, rv, sm = bufs[t % 2]
            handles[t].wait()
            pltpu.sync_copy(rv, out_hbm.at[pl.ds(base + t * cs, cs)])
            nt = t + 2
            if nt < nchunk:
                niv, nrv, nsm = bufs[nt % 2]
                pltpu.sync_copy(idx_hbm.at[pl.ds(base + nt * cs, cs)], niv)
                handles[nt] = pltpu.async_copy(table_hbm.at[niv], nrv, nsm)

    return gather


def _recon_body(z_ref, q_ref, wd_ref, bd_ref, qst_ref, rec_ref, com_ref):
    i = pl.program_id(0)
    z = z_ref[...]
    q = q_ref[...]
    qst = z + (q - z)
    qst_ref[...] = qst
    rec_ref[...] = _dot256(qst, wd_ref[...]) + bd_ref[...]

    @pl.when(i == 0)
    def _():
        com_ref[...] = jnp.zeros((1, 1), jnp.float32)

    com_ref[...] += jnp.sum((z - q) ** 2, keepdims=True)

    @pl.when(i == pl.num_programs(0) - 1)
    def _():
        com_ref[...] = com_ref[...] / (_B * _CD)


@functools.lru_cache(maxsize=None)
def _recon_call(d):
    return pl.pallas_call(
        _recon_body,
        grid=(_B // _RB,),
        in_specs=[pl.BlockSpec((_RB, _CD), lambda i: (i, 0)),
                  pl.BlockSpec((_RB, _CD), lambda i: (i, 0)),
                  pl.BlockSpec((d, _CD), lambda i: (0, 0)),
                  pl.BlockSpec((1, d), lambda i: (0, 0))],
        out_specs=[pl.BlockSpec((_RB, _CD), lambda i: (i, 0)),
                   pl.BlockSpec((_RB, d), lambda i: (i, 0)),
                   pl.BlockSpec((1, 1), lambda i: (0, 0))],
        out_shape=[jax.ShapeDtypeStruct((_B, _CD), jnp.float32),
                   jax.ShapeDtypeStruct((_B, d), jnp.float32),
                   jax.ShapeDtypeStruct((1, 1), jnp.float32)],
    )


def kernel(LM, VIS, CLIP, MAE, params):
    xs = {'LM': LM, 'VIS': VIS, 'CLIP': CLIP, 'MAE': MAE}
    E = params['codebook']
    gather = _get_gather()
    esq = _esq_call(E)

    h1s = [_mm_call(_DIMS[m], _H)(xs[m], params[m]['W1'],
                                  params[m]['b1'].reshape(1, _H))
           for m in _MODS]
    H1 = jnp.concatenate(h1s, axis=0)
    st = lambda name, n: jnp.stack(
        [params[m][name].reshape(1, n) for m in _MODS])
    Z, idx2 = _mid4_call(H1, st('g1', _H), st('be1', _H),
                         jnp.stack([params[m]['W2'] for m in _MODS]),
                         st('b2', _CD), st('g2', _CD), st('be2', _CD),
                         E, esq)
    idx_all = idx2.reshape(4 * _B)
    Q = gather(E, idx_all)

    out = {}
    for i, m in enumerate(_MODS):
        p = params[m]
        d = _DIMS[m]
        z = Z[i * _B:(i + 1) * _B]
        q = Q[i * _B:(i + 1) * _B]
        qst, rec, com = _recon_call(d)(z, q, p['Wd'], p['bd'].reshape(1, d))
        out[f'{m}_z'] = z
        out[f'{m}_q'] = qst
        out[f'{m}_idx'] = idx_all[i * _B:(i + 1) * _B]
        out[f'{m}_commit'] = com.reshape(())
        out[f'{m}_recon'] = rec
    return out
